# Initial kernel scaffold; baseline (speedup 1.0000x reference)
import jax
import jax.numpy as jnp
from jax.experimental import pallas as pl


def kernel(pv_y_osgb_fourier, pv_x_osgb_fourier, pv_system_row_number, pv_time_utc_fourier, pv_x_osgb, solar_azimuth, solar_elevation, pv_embedding):
    raise NotImplementedError("write your pallas kernel here")


# speedup vs baseline: 1.3862x; 1.3862x over previous
import functools

import jax
import jax.numpy as jnp
from jax import lax
from jax.experimental import pallas as pl
from jax.experimental.pallas import tpu as pltpu
from jax.experimental.pallas import tpu_sc as plsc

_B = 256
_N_PV = 1400
_F = 64
_EMB = 32
_OUTC = 2 * _F + _F + _EMB + 2
_NROWS = _B * _N_PV

_NC = 2
_NS = 16
_NW = _NC * _NS
_BPW = _NROWS // _NW
_CHUNK = 2240
_NCHUNKS = _BPW // _CHUNK


def _sc_gather(table, idx):
    mesh = plsc.VectorSubcoreMesh(core_axis_name="c", subcore_axis_name="s")

    @functools.partial(
        pl.kernel,
        mesh=mesh,
        compiler_params=pltpu.CompilerParams(use_tc_tiling_on_sc=False),
        out_type=jax.ShapeDtypeStruct((_NROWS, _EMB), jnp.float32),
        scratch_types=[
            pltpu.VMEM((_CHUNK,), jnp.int32),
            pltpu.VMEM((_CHUNK, _EMB), jnp.float32),
            pltpu.SemaphoreType.DMA,
        ],
    )
    def k(table_hbm, idx_hbm, out_hbm, idx_v, rows_v, sem):
        wid = lax.axis_index("s") * _NC + lax.axis_index("c")
        base = wid * _BPW

        def body(i, carry):
            off = base + i * _CHUNK
            pltpu.sync_copy(idx_hbm.at[pl.ds(off, _CHUNK)], idx_v)
            pltpu.async_copy(table_hbm.at[idx_v], rows_v, sem).wait()
            pltpu.sync_copy(rows_v, out_hbm.at[pl.ds(off, _CHUNK)])
            return carry

        lax.fori_loop(0, _NCHUNKS, body, 0)

    return k(table, idx)


def _fix(v):
    return jnp.nan_to_num(v)


def _tc_body(y_ref, x_ref, t_ref, emb_ref, az_ref, el_ref, out_ref):
    b = pl.program_id(0)
    y = _fix(y_ref[0])
    x = _fix(x_ref[0])
    t = _fix(t_ref[0, 0])
    e = _fix(emb_ref[0])
    az = _fix(az_ref[b])
    el = _fix(el_ref[b])
    tb = jnp.broadcast_to(t[None, :], (_N_PV, _F))
    azc = jnp.full((_N_PV, 1), az, jnp.float32)
    elc = jnp.full((_N_PV, 1), el, jnp.float32)
    out_ref[0] = jnp.concatenate([y, x, tb, e, azc, elc], axis=-1)


def _tc_assemble(y, x, t, emb, az, el):
    return pl.pallas_call(
        _tc_body,
        grid=(_B,),
        in_specs=[
            pl.BlockSpec((1, _N_PV, _F), lambda b: (b, 0, 0)),
            pl.BlockSpec((1, _N_PV, _F), lambda b: (b, 0, 0)),
            pl.BlockSpec((1, 1, _F), lambda b: (b, 0, 0)),
            pl.BlockSpec((1, _N_PV, _EMB), lambda b: (b, 0, 0)),
            pl.BlockSpec(memory_space=pltpu.SMEM),
            pl.BlockSpec(memory_space=pltpu.SMEM),
        ],
        out_specs=pl.BlockSpec((1, _N_PV, _OUTC), lambda b: (b, 0, 0)),
        out_shape=jax.ShapeDtypeStruct((_B, _N_PV, _OUTC), jnp.float32),
    )(y, x, t, emb, az, el)


def kernel(pv_y_osgb_fourier, pv_x_osgb_fourier, pv_system_row_number,
           pv_time_utc_fourier, pv_x_osgb, solar_azimuth, solar_elevation,
           pv_embedding):
    idx_flat = pv_system_row_number.reshape(-1).astype(jnp.int32)
    emb_rows = _sc_gather(pv_embedding, idx_flat)
    emb = emb_rows.reshape(_B, _N_PV, _EMB)
    t = pv_time_utc_fourier[:, 12][:, None, :]
    az = solar_azimuth[:, 12]
    el = solar_elevation[:, 12]
    return _tc_assemble(pv_y_osgb_fourier, pv_x_osgb_fourier, t, emb, az, el)
